# baseline (speedup 1.0000x reference)
import jax
import jax.numpy as jnp
from jax.experimental import pallas as pl
from jax.experimental.pallas import tpu as pltpu


def _plan(B, D, K):
    BT = min(256, B)
    if D % 128 == 0 or D < 1280:
        DC = min(D, 1280)
    else:
        DC = 1280
    ND = -(-D // DC)
    return BT, DC, ND


def _body(ND, D, DC, qw_ref, pv_ref, w_ref, lb_ref, scores_ref, pred_ref,
          q_scr, qss_scr, lss_scr):
    i = pl.program_id(0)
    d = pl.program_id(1)

    def _col_mask(rows, chunk):
        col = jax.lax.broadcasted_iota(jnp.int32, (rows, DC), 1)
        return col < (D - chunk * DC)

    @pl.when(d < ND)
    def _phase1():
        q = jnp.tanh(qw_ref[...] * w_ref[0:1, :] + pv_ref[...] * w_ref[1:2, :])
        q = jnp.where(_col_mask(q.shape[0], d), q, 0.0)
        q_scr[d] = q
        ss = jnp.sum(q * q, axis=1, keepdims=True)

        @pl.when(d == 0)
        def _():
            qss_scr[...] = ss

        @pl.when(d > 0)
        def _():
            qss_scr[...] += ss

    @pl.when((d < ND) & (i == 0))
    def _label_ss():
        lb = jnp.where(_col_mask(lb_ref.shape[0], d), lb_ref[...], 0.0)
        ls = jnp.sum(lb * lb, axis=1, keepdims=True)

        @pl.when(d == 0)
        def _():
            lss_scr[...] = ls

        @pl.when(d > 0)
        def _():
            lss_scr[...] += ls

    @pl.when(d >= ND)
    def _phase2():
        dd = d - ND
        rnq = 1.0 / (jnp.sqrt(qss_scr[...]) + 1e-8)
        rnl = 1.0 / (jnp.sqrt(lss_scr[...]) + 1e-8)
        qn = (q_scr[dd] * rnq).astype(jnp.bfloat16)
        lb = jnp.where(_col_mask(lb_ref.shape[0], dd), lb_ref[...], 0.0)
        ln = (lb * rnl).astype(jnp.bfloat16)
        part = jax.lax.dot_general(
            qn, ln, (((1,), (1,)), ((), ())),
            preferred_element_type=jnp.float32)

        @pl.when(dd == 0)
        def _():
            scores_ref[...] = part

        @pl.when(dd > 0)
        def _():
            scores_ref[...] += part

        @pl.when(dd == ND - 1)
        def _():
            pred_ref[...] = jnp.argmax(
                scores_ref[...], axis=1, keepdims=True).astype(jnp.int32)


def kernel(q_word, pvs, query_weight, label):
    B, D = q_word.shape
    K = label.shape[0]
    BT, DC, ND = _plan(B, D, K)
    NB = B // BT
    import functools
    body = functools.partial(_body, ND, D, DC)
    grid = (NB, 2 * ND)
    scores, pred = pl.pallas_call(
        body,
        grid=grid,
        in_specs=[
            pl.BlockSpec((BT, DC), lambda i, d: (i, jnp.minimum(d, ND - 1))),
            pl.BlockSpec((BT, DC), lambda i, d: (i, jnp.minimum(d, ND - 1))),
            pl.BlockSpec((2, DC), lambda i, d: (0, jnp.minimum(d, ND - 1))),
            pl.BlockSpec((K, DC), lambda i, d: (
                0, jnp.where(d >= ND, d - ND, jnp.where(i == 0, d, 0)))),
        ],
        out_specs=[
            pl.BlockSpec((BT, K), lambda i, d: (i, 0)),
            pl.BlockSpec((BT, 1), lambda i, d: (i, 0)),
        ],
        out_shape=[
            jax.ShapeDtypeStruct((B, K), jnp.float32),
            jax.ShapeDtypeStruct((B, 1), jnp.int32),
        ],
        scratch_shapes=[
            pltpu.VMEM((ND, BT, DC), jnp.float32),
            pltpu.VMEM((BT, 1), jnp.float32),
            pltpu.VMEM((K, 1), jnp.float32),
        ],
    )(q_word, pvs, query_weight, label)
    return scores, pred.reshape(B)


# speedup vs baseline: 1.3656x; 1.3656x over previous
import functools

import jax
import jax.numpy as jnp
from jax.experimental import pallas as pl
from jax.experimental.pallas import tpu as pltpu


def _plan(B, D, K):
    BT = min(256, B)
    DC = min(D, 1280)
    ND = -(-D // DC)
    return BT, DC, ND


def _body(NB, ND, D, DC, qw_ref, pv_ref, w_ref, lt_ref, scores_ref, pred_ref,
          q_scr, ln_scr, qss_scr, rnq_scr, lss_scr, rnl_scr):
    s = pl.program_id(0)
    d = pl.program_id(1)

    def _col_mask(rows):
        col = jax.lax.broadcasted_iota(jnp.int32, (rows, DC), 1)
        return col < (D - d * DC)

    def _row_mask(cols):
        row = jax.lax.broadcasted_iota(jnp.int32, (DC, cols), 0)
        return row < (D - d * DC)

    @pl.when(s == 0)
    def _label_ss():
        lt = jnp.where(_row_mask(lt_ref.shape[1]), lt_ref[...], 0.0)
        ls = jnp.sum(lt * lt, axis=0, keepdims=True)

        @pl.when(d == 0)
        def _():
            lss_scr[...] = ls

        @pl.when(d > 0)
        def _():
            lss_scr[...] += ls

    @pl.when((s == 1) & (d == 0))
    def _latch_rnl():
        rnl_scr[...] = 1.0 / (jnp.sqrt(lss_scr[...]) + 1e-8)

    @pl.when((s >= 1) & (d == 0))
    def _latch_rnq():
        rnq_scr[...] = 1.0 / (jnp.sqrt(qss_scr[...]) + 1e-8)

    @pl.when(s >= 1)
    def _matmul_prev():
        @pl.when(s == 1)
        def _ln_fresh():
            lt = jnp.where(_row_mask(lt_ref.shape[1]), lt_ref[...], 0.0)
            ln_scr[d] = (lt * rnl_scr[...]).astype(jnp.bfloat16)

        qn = (q_scr[d] * rnq_scr[...]).astype(jnp.bfloat16)
        ln = ln_scr[d]
        part = jax.lax.dot_general(
            qn, ln, (((1,), (0,)), ((), ())),
            preferred_element_type=jnp.float32)

        @pl.when(d == 0)
        def _():
            scores_ref[...] = part

        @pl.when(d > 0)
        def _():
            scores_ref[...] += part

        @pl.when(d == ND - 1)
        def _():
            pred_ref[...] = jnp.argmax(
                scores_ref[...], axis=1, keepdims=True).astype(jnp.int32)

    @pl.when(s < NB)
    def _build_q():
        q = jnp.tanh(qw_ref[...] * w_ref[0:1, :] + pv_ref[...] * w_ref[1:2, :])
        q = jnp.where(_col_mask(q.shape[0]), q, 0.0)
        q_scr[d] = q
        ss = jnp.sum(q * q, axis=1, keepdims=True)

        @pl.when(d == 0)
        def _():
            qss_scr[...] = ss

        @pl.when(d > 0)
        def _():
            qss_scr[...] += ss


def kernel(q_word, pvs, query_weight, label):
    B, D = q_word.shape
    K = label.shape[0]
    BT, DC, ND = _plan(B, D, K)
    NB = B // BT
    label_t = jnp.swapaxes(label, 0, 1)
    body = functools.partial(_body, NB, ND, D, DC)
    grid = (NB + 1, ND)
    scores, pred = pl.pallas_call(
        body,
        grid=grid,
        in_specs=[
            pl.BlockSpec((BT, DC), lambda s, d: (
                jnp.minimum(s, NB - 1), jnp.where(s < NB, d, ND - 1))),
            pl.BlockSpec((BT, DC), lambda s, d: (
                jnp.minimum(s, NB - 1), jnp.where(s < NB, d, ND - 1))),
            pl.BlockSpec((2, DC), lambda s, d: (0, jnp.where(s < NB, d, ND - 1))),
            pl.BlockSpec((DC, K), lambda s, d: (
                jnp.where(s <= 1, d, ND - 1), 0)),
        ],
        out_specs=[
            pl.BlockSpec((BT, K), lambda s, d: (jnp.maximum(s - 1, 0), 0)),
            pl.BlockSpec((BT, 1), lambda s, d: (jnp.maximum(s - 1, 0), 0)),
        ],
        out_shape=[
            jax.ShapeDtypeStruct((B, K), jnp.float32),
            jax.ShapeDtypeStruct((B, 1), jnp.int32),
        ],
        scratch_shapes=[
            pltpu.VMEM((ND, BT, DC), jnp.float32),
            pltpu.VMEM((ND, DC, K), jnp.bfloat16),
            pltpu.VMEM((BT, 1), jnp.float32),
            pltpu.VMEM((BT, 1), jnp.float32),
            pltpu.VMEM((1, K), jnp.float32),
            pltpu.VMEM((1, K), jnp.float32),
        ],
    )(q_word, pvs, query_weight, label_t)
    return scores, pred.reshape(B)
